# Initial kernel scaffold; baseline (speedup 1.0000x reference)
import functools

import jax
import jax.numpy as jnp
from jax import lax
from jax.experimental import pallas as pl
from jax.experimental.pallas import tpu as pltpu
from jax.experimental.pallas import tpu_sc as plsc

N = 10000
E = 320000
D = 128

NC = 2
NS = 16
NW = NC * NS
B = 128
NCHUNK = -(-E // (NW * B))
E_PAD = NW * B * NCHUNK
N_PAD = 10112
ROWS_PT = N_PAD // NS

_mesh = plsc.VectorSubcoreMesh(
    core_axis_name="c", subcore_axis_name="s", num_cores=NC, num_subcores=NS
)


@functools.partial(
    pl.kernel,
    out_type=jax.ShapeDtypeStruct((NC, N_PAD), jnp.float32),
    mesh=_mesh,
    scratch_types=[
        pltpu.VMEM((NCHUNK, B), jnp.int32),
        pltpu.VMEM((B,), jnp.float32),
        pltpu.VMEM_SHARED((N_PAD,), jnp.float32),
    ],
)
def _sc_degree(dst_hbm, ones_hbm, zeros_hbm, out_hbm, dst_v, ones_v, deg_sh):
    c = lax.axis_index("c")
    s = lax.axis_index("s")
    w = c * NS + s
    pltpu.sync_copy(
        zeros_hbm.at[pl.ds(s * ROWS_PT, ROWS_PT)],
        deg_sh.at[pl.ds(s * ROWS_PT, ROWS_PT)],
    )
    pltpu.sync_copy(dst_hbm.at[pl.ds(w * NCHUNK, NCHUNK)], dst_v)
    pltpu.sync_copy(ones_hbm, ones_v)
    plsc.subcore_barrier()

    def body(j, carry):
        pltpu.sync_copy(ones_v, deg_sh.at[dst_v.at[j]], add=True)
        return carry

    lax.fori_loop(0, NCHUNK, body, 0)
    plsc.subcore_barrier()
    pltpu.sync_copy(
        deg_sh.at[pl.ds(s * ROWS_PT, ROWS_PT)],
        out_hbm.at[c, pl.ds(s * ROWS_PT, ROWS_PT)],
    )


@functools.partial(
    pl.kernel,
    out_type=jax.ShapeDtypeStruct((NC, N_PAD, D), jnp.float32),
    mesh=_mesh,
    scratch_types=[
        pltpu.VMEM((NCHUNK, B), jnp.int32),
        pltpu.VMEM((NCHUNK, B), jnp.int32),
        pltpu.VMEM((B, D), jnp.float32),
        pltpu.VMEM_SHARED((N_PAD, D), jnp.float32),
        pltpu.SemaphoreType.DMA,
    ],
)
def _sc_scatter(hp_hbm, src_hbm, dst_hbm, zeros_hbm, out_hbm,
                src_v, dst_v, rows_v, agg_sh, sem):
    c = lax.axis_index("c")
    s = lax.axis_index("s")
    w = c * NS + s
    pltpu.sync_copy(
        zeros_hbm.at[pl.ds(s * ROWS_PT, ROWS_PT)],
        agg_sh.at[pl.ds(s * ROWS_PT, ROWS_PT)],
    )
    pltpu.sync_copy(src_hbm.at[pl.ds(w * NCHUNK, NCHUNK)], src_v)
    pltpu.sync_copy(dst_hbm.at[pl.ds(w * NCHUNK, NCHUNK)], dst_v)
    plsc.subcore_barrier()

    def body(j, carry):
        pltpu.async_copy(hp_hbm.at[src_v.at[j]], rows_v, sem).wait()
        pltpu.sync_copy(rows_v, agg_sh.at[dst_v.at[j]], add=True)
        return carry

    lax.fori_loop(0, NCHUNK, body, 0)
    plsc.subcore_barrier()
    pltpu.sync_copy(
        agg_sh.at[pl.ds(s * ROWS_PT, ROWS_PT)],
        out_hbm.at[c, pl.ds(s * ROWS_PT, ROWS_PT)],
    )


def _scale_body(x_ref, w_ref, degT_ref, hp_ref, dinv_ref):
    deg = degT_ref[:, 0:1] + degT_ref[:, 1:2] + 1.0
    dinv = lax.rsqrt(deg)
    h = jnp.dot(x_ref[...], w_ref[...], preferred_element_type=jnp.float32)
    hp_ref[...] = h * dinv
    dinv_ref[...] = dinv


_scale = pl.pallas_call(
    _scale_body,
    out_shape=(
        jax.ShapeDtypeStruct((N_PAD, D), jnp.float32),
        jax.ShapeDtypeStruct((N_PAD, 1), jnp.float32),
    ),
)

RB = 400


def _epilogue_body(sp_ref, hp_ref, dinv_ref, x_ref, b_ref, g_ref, be_ref, out_ref):
    agg = (sp_ref[0] + sp_ref[1] + hp_ref[...]) * dinv_ref[...] + b_ref[...]
    mu = jnp.mean(agg, axis=-1, keepdims=True)
    cen = agg - mu
    var = jnp.mean(cen * cen, axis=-1, keepdims=True)
    ln = cen * lax.rsqrt(var + 1e-5) * g_ref[...] + be_ref[...]
    o = ln + x_ref[...]
    out_ref[...] = o * (1.0 / (1.0 + jnp.exp(-o)))


_epilogue = pl.pallas_call(
    _epilogue_body,
    grid=(N // RB,),
    in_specs=[
        pl.BlockSpec((NC, RB, D), lambda i: (0, i, 0)),
        pl.BlockSpec((RB, D), lambda i: (i, 0)),
        pl.BlockSpec((RB, 1), lambda i: (i, 0)),
        pl.BlockSpec((RB, D), lambda i: (i, 0)),
        pl.BlockSpec((1, D), lambda i: (0, 0)),
        pl.BlockSpec((1, D), lambda i: (0, 0)),
        pl.BlockSpec((1, D), lambda i: (0, 0)),
    ],
    out_specs=pl.BlockSpec((RB, D), lambda i: (i, 0)),
    out_shape=jax.ShapeDtypeStruct((N, D), jnp.float32),
)


def kernel(x, edge_index, W, b, gamma, beta):
    pad_idx = jnp.full((E_PAD - E,), N, dtype=jnp.int32)
    src2d = jnp.concatenate([edge_index[0], pad_idx]).reshape(NW * NCHUNK, B)
    dst2d = jnp.concatenate([edge_index[1], pad_idx]).reshape(NW * NCHUNK, B)
    x_pad = jnp.concatenate([x, jnp.zeros((N_PAD - N, D), x.dtype)], axis=0)
    ones_row = jnp.ones((B,), jnp.float32)
    zeros1 = jnp.zeros((N_PAD,), jnp.float32)
    zeros2 = jnp.zeros((N_PAD, D), jnp.float32)

    degp = _sc_degree(dst2d, ones_row, zeros1)
    degT = jnp.transpose(degp)
    hp, dinv = _scale(x_pad, W, degT)
    sp = _sc_scatter(hp, src2d, dst2d, zeros2)
    return _epilogue(
        sp[:, :N, :], hp[:N], dinv[:N], x,
        b.reshape(1, D), gamma.reshape(1, D), beta.reshape(1, D),
    )


# speedup vs baseline: 12.6793x; 12.6793x over previous
import functools

import jax
import jax.numpy as jnp
from jax import lax
from jax.experimental import pallas as pl
from jax.experimental.pallas import tpu as pltpu
from jax.experimental.pallas import tpu_sc as plsc

N = 10000
E = 320000
D = 128

NC = 2
NS = 16
NW = NC * NS
B = 128
NCHUNK = (-(-E // (NW * B)) + 7) // 8 * 8
E_PAD = NW * B * NCHUNK
N_PAD = 10240
ROWS_PT = N_PAD // NS

_mesh = plsc.VectorSubcoreMesh(
    core_axis_name="c", subcore_axis_name="s", num_cores=NC, num_subcores=NS
)


@functools.partial(
    pl.kernel,
    out_type=jax.ShapeDtypeStruct((NC, 1, N_PAD), jnp.float32),
    mesh=_mesh,
    scratch_types=[
        pltpu.VMEM((NCHUNK, B), jnp.int32),
        pltpu.VMEM((B,), jnp.float32),
        pltpu.VMEM_SHARED((N_PAD,), jnp.float32),
    ],
)
def _sc_degree(dst_hbm, ones_hbm, zeros_hbm, out_hbm, dst_v, ones_v, deg_sh):
    c = lax.axis_index("c")
    s = lax.axis_index("s")
    w = c * NS + s
    pltpu.sync_copy(
        zeros_hbm.at[pl.ds(s * ROWS_PT, ROWS_PT)],
        deg_sh.at[pl.ds(s * ROWS_PT, ROWS_PT)],
    )
    pltpu.sync_copy(dst_hbm.at[pl.ds(w * NCHUNK, NCHUNK)], dst_v)
    pltpu.sync_copy(ones_hbm, ones_v)
    plsc.subcore_barrier()

    def body(j, carry):
        pltpu.sync_copy(ones_v, deg_sh.at[dst_v.at[j]], add=True)
        return carry

    lax.fori_loop(0, NCHUNK, body, 0)
    plsc.subcore_barrier()
    pltpu.sync_copy(
        deg_sh.at[pl.ds(s * ROWS_PT, ROWS_PT)],
        out_hbm.at[c, 0, pl.ds(s * ROWS_PT, ROWS_PT)],
    )


@functools.partial(
    pl.kernel,
    out_type=jax.ShapeDtypeStruct((NC, N_PAD, D), jnp.float32),
    mesh=_mesh,
    scratch_types=[
        pltpu.VMEM((NCHUNK, B), jnp.int32),
        pltpu.VMEM((NCHUNK, B), jnp.int32),
        pltpu.VMEM((B, D), jnp.float32),
        pltpu.VMEM_SHARED((N_PAD, D), jnp.float32),
        pltpu.SemaphoreType.DMA,
    ],
)
def _sc_scatter(hp_hbm, src_hbm, dst_hbm, zeros_hbm, out_hbm,
                src_v, dst_v, rows_v, agg_sh, sem):
    c = lax.axis_index("c")
    s = lax.axis_index("s")
    w = c * NS + s
    pltpu.sync_copy(
        zeros_hbm.at[pl.ds(s * ROWS_PT, ROWS_PT)],
        agg_sh.at[pl.ds(s * ROWS_PT, ROWS_PT)],
    )
    pltpu.sync_copy(src_hbm.at[pl.ds(w * NCHUNK, NCHUNK)], src_v)
    pltpu.sync_copy(dst_hbm.at[pl.ds(w * NCHUNK, NCHUNK)], dst_v)
    plsc.subcore_barrier()

    def body(j, carry):
        pltpu.async_copy(hp_hbm.at[src_v.at[j]], rows_v, sem).wait()
        pltpu.sync_copy(rows_v, agg_sh.at[dst_v.at[j]], add=True)
        return carry

    lax.fori_loop(0, NCHUNK, body, 0)
    plsc.subcore_barrier()
    pltpu.sync_copy(
        agg_sh.at[pl.ds(s * ROWS_PT, ROWS_PT)],
        out_hbm.at[c, pl.ds(s * ROWS_PT, ROWS_PT)],
    )


def _scale_body(x_ref, w_ref, degT_ref, hp_ref, dinv_ref):
    deg = degT_ref[:, 0:1] + degT_ref[:, 1:2] + 1.0
    dinv = lax.rsqrt(deg)
    h = jnp.dot(x_ref[...], w_ref[...], preferred_element_type=jnp.float32)
    hp_ref[...] = h * dinv
    dinv_ref[...] = dinv


_scale = pl.pallas_call(
    _scale_body,
    out_shape=(
        jax.ShapeDtypeStruct((N_PAD, D), jnp.float32),
        jax.ShapeDtypeStruct((N_PAD, 1), jnp.float32),
    ),
)

RB = 400


def _epilogue_body(sp_ref, hp_ref, dinv_ref, x_ref, b_ref, g_ref, be_ref, out_ref):
    agg = (sp_ref[0] + sp_ref[1] + hp_ref[...]) * dinv_ref[...] + b_ref[...]
    mu = jnp.mean(agg, axis=-1, keepdims=True)
    cen = agg - mu
    var = jnp.mean(cen * cen, axis=-1, keepdims=True)
    ln = cen * lax.rsqrt(var + 1e-5) * g_ref[...] + be_ref[...]
    o = ln + x_ref[...]
    out_ref[...] = o * (1.0 / (1.0 + jnp.exp(-o)))


_epilogue = pl.pallas_call(
    _epilogue_body,
    grid=(N // RB,),
    in_specs=[
        pl.BlockSpec((NC, RB, D), lambda i: (0, i, 0)),
        pl.BlockSpec((RB, D), lambda i: (i, 0)),
        pl.BlockSpec((RB, 1), lambda i: (i, 0)),
        pl.BlockSpec((RB, D), lambda i: (i, 0)),
        pl.BlockSpec((1, D), lambda i: (0, 0)),
        pl.BlockSpec((1, D), lambda i: (0, 0)),
        pl.BlockSpec((1, D), lambda i: (0, 0)),
    ],
    out_specs=pl.BlockSpec((RB, D), lambda i: (i, 0)),
    out_shape=jax.ShapeDtypeStruct((N, D), jnp.float32),
)


def kernel(x, edge_index, W, b, gamma, beta):
    pad_idx = jnp.full((E_PAD - E,), N, dtype=jnp.int32)
    src2d = jnp.concatenate([edge_index[0], pad_idx]).reshape(NW * NCHUNK, B)
    dst2d = jnp.concatenate([edge_index[1], pad_idx]).reshape(NW * NCHUNK, B)
    x_pad = jnp.concatenate([x, jnp.zeros((N_PAD - N, D), x.dtype)], axis=0)
    ones_row = jnp.ones((B,), jnp.float32)
    zeros1 = jnp.zeros((N_PAD,), jnp.float32)
    zeros2 = jnp.zeros((N_PAD, D), jnp.float32)

    degp = _sc_degree(dst2d, ones_row, zeros1)
    degT = jnp.transpose(degp.reshape(NC, N_PAD))
    hp, dinv = _scale(x_pad, W, degT)
    sp = _sc_scatter(hp, src2d, dst2d, zeros2)
    return _epilogue(
        sp[:, :N, :], hp[:N], dinv[:N], x,
        b.reshape(1, D), gamma.reshape(1, D), beta.reshape(1, D),
    )


# speedup vs baseline: 15.1822x; 1.1974x over previous
import functools

import jax
import jax.numpy as jnp
from jax import lax
from jax.experimental import pallas as pl
from jax.experimental.pallas import tpu as pltpu
from jax.experimental.pallas import tpu_sc as plsc

N = 10000
E = 320000
D = 128
DH = D // 2

NC = 2
NS = 16
NW = NC * NS
B = 128
CHP = 2560
E_PAD = CHP * B
NCHUNK = CHP // NS
N_PAD = 10240
ROWS_PT = N_PAD // NS
NBUF = 4
NGRP = NCHUNK // NBUF

_mesh = plsc.VectorSubcoreMesh(
    core_axis_name="c", subcore_axis_name="s", num_cores=NC, num_subcores=NS
)


@functools.partial(
    pl.kernel,
    out_type=jax.ShapeDtypeStruct((NC, 1, N_PAD), jnp.float32),
    mesh=_mesh,
    scratch_types=[
        pltpu.VMEM((NCHUNK // 2, B), jnp.int32),
        pltpu.VMEM((B,), jnp.float32),
        pltpu.VMEM_SHARED((N_PAD,), jnp.float32),
        pltpu.SemaphoreType.DMA,
    ],
)
def _sc_degree(dst_hbm, ones_hbm, zeros_hbm, out_hbm, dst_v, ones_v, deg_sh, sem):
    c = lax.axis_index("c")
    s = lax.axis_index("s")
    w = c * NS + s
    nch = NCHUNK // 2
    pltpu.sync_copy(
        zeros_hbm.at[pl.ds(s * ROWS_PT, ROWS_PT)],
        deg_sh.at[pl.ds(s * ROWS_PT, ROWS_PT)],
    )
    pltpu.sync_copy(dst_hbm.at[pl.ds(w * nch, nch)], dst_v)
    pltpu.sync_copy(ones_hbm, ones_v)
    plsc.subcore_barrier()

    def fire(j, carry):
        pltpu.async_copy(ones_v, deg_sh.at[dst_v.at[j]], sem, add=True)
        return carry

    lax.fori_loop(0, nch, fire, 0)

    def drain(j, carry):
        pltpu.make_async_copy(ones_v, deg_sh.at[dst_v.at[j]], sem).wait()
        return carry

    lax.fori_loop(0, nch, drain, 0)
    plsc.subcore_barrier()
    pltpu.sync_copy(
        deg_sh.at[pl.ds(s * ROWS_PT, ROWS_PT)],
        out_hbm.at[c, 0, pl.ds(s * ROWS_PT, ROWS_PT)],
    )


@functools.partial(
    pl.kernel,
    out_type=jax.ShapeDtypeStruct((NC, N_PAD, DH), jnp.float32),
    mesh=_mesh,
    scratch_types=[
        pltpu.VMEM((NCHUNK, B), jnp.int32),
        pltpu.VMEM((NCHUNK, B), jnp.int32),
        pltpu.VMEM((NBUF, B, DH), jnp.float32),
        pltpu.VMEM_SHARED((N_PAD, DH), jnp.float32),
        pltpu.SemaphoreType.DMA((NBUF,)),
        pltpu.SemaphoreType.DMA((NBUF,)),
    ],
    compiler_params=pltpu.CompilerParams(use_tc_tiling_on_sc=False),
)
def _sc_scatter(hp2_hbm, gidx_hbm, dst_hbm, zeros_hbm, out_hbm,
                src_v, dst_v, rows_v, agg_sh, gsem, ssem):
    c = lax.axis_index("c")
    s = lax.axis_index("s")
    pltpu.sync_copy(
        zeros_hbm.at[pl.ds(s * ROWS_PT, ROWS_PT)],
        agg_sh.at[pl.ds(s * ROWS_PT, ROWS_PT)],
    )
    pltpu.sync_copy(gidx_hbm.at[c, pl.ds(s * NCHUNK, NCHUNK)], src_v)
    pltpu.sync_copy(dst_hbm.at[pl.ds(s * NCHUNK, NCHUNK)], dst_v)
    plsc.subcore_barrier()

    for b in range(NBUF):
        pltpu.async_copy(hp2_hbm.at[src_v.at[b]], rows_v.at[b], gsem.at[b])

    def group(g, carry):
        base = g * NBUF
        for b in range(NBUF):
            pltpu.make_async_copy(
                hp2_hbm.at[src_v.at[base + b]], rows_v.at[b], gsem.at[b]
            ).wait()
            pltpu.async_copy(
                rows_v.at[b], agg_sh.at[dst_v.at[base + b]], ssem.at[b], add=True
            )
        for b in range(NBUF):
            pltpu.make_async_copy(
                rows_v.at[b], agg_sh.at[dst_v.at[base + b]], ssem.at[b]
            ).wait()
            nxt = jnp.minimum(base + NBUF + b, NCHUNK - 1)

            @pl.when(g + 1 < NGRP)
            def _():
                pltpu.async_copy(hp2_hbm.at[src_v.at[nxt]], rows_v.at[b], gsem.at[b])

        return carry

    lax.fori_loop(0, NGRP, group, 0)
    plsc.subcore_barrier()
    pltpu.sync_copy(
        agg_sh.at[pl.ds(s * ROWS_PT, ROWS_PT)],
        out_hbm.at[c, pl.ds(s * ROWS_PT, ROWS_PT)],
    )


def _scale_body(x_ref, w_ref, degT_ref, hp_ref, dinv_ref):
    deg = degT_ref[:, 0:1] + degT_ref[:, 1:2] + 1.0
    dinv = lax.rsqrt(deg)
    h = jnp.dot(x_ref[...], w_ref[...], preferred_element_type=jnp.float32)
    hp_ref[...] = h * dinv
    dinv_ref[...] = dinv


_scale = pl.pallas_call(
    _scale_body,
    out_shape=(
        jax.ShapeDtypeStruct((N, D), jnp.float32),
        jax.ShapeDtypeStruct((N, 1), jnp.float32),
    ),
)

RB = 400


def _epilogue_body(sp_ref, hp_ref, dinv_ref, x_ref, b_ref, g_ref, be_ref, out_ref):
    ssum = jnp.concatenate([sp_ref[0], sp_ref[1]], axis=-1)
    agg = (ssum + hp_ref[...]) * dinv_ref[...] + b_ref[...]
    mu = jnp.mean(agg, axis=-1, keepdims=True)
    cen = agg - mu
    var = jnp.mean(cen * cen, axis=-1, keepdims=True)
    ln = cen * lax.rsqrt(var + 1e-5) * g_ref[...] + be_ref[...]
    o = ln + x_ref[...]
    out_ref[...] = o * (1.0 / (1.0 + jnp.exp(-o)))


_epilogue = pl.pallas_call(
    _epilogue_body,
    grid=(N // RB,),
    in_specs=[
        pl.BlockSpec((NC, RB, DH), lambda i: (0, i, 0)),
        pl.BlockSpec((RB, D), lambda i: (i, 0)),
        pl.BlockSpec((RB, 1), lambda i: (i, 0)),
        pl.BlockSpec((RB, D), lambda i: (i, 0)),
        pl.BlockSpec((1, D), lambda i: (0, 0)),
        pl.BlockSpec((1, D), lambda i: (0, 0)),
        pl.BlockSpec((1, D), lambda i: (0, 0)),
    ],
    out_specs=pl.BlockSpec((RB, D), lambda i: (i, 0)),
    out_shape=jax.ShapeDtypeStruct((N, D), jnp.float32),
)


def kernel(x, edge_index, W, b, gamma, beta):
    npad = E_PAD - E
    src_pad = jnp.concatenate([edge_index[0], jnp.zeros((npad,), jnp.int32)])
    dst_fill = N + (jnp.arange(npad, dtype=jnp.int32) % (N_PAD - N))
    dst_pad = jnp.concatenate([edge_index[1], dst_fill])
    src2 = src_pad * 2
    gidx = jnp.stack([src2, src2 + 1]).reshape(NC, CHP, B)
    dst2d = dst_pad.reshape(CHP, B)
    ones_row = jnp.ones((B,), jnp.float32)
    zeros1 = jnp.zeros((N_PAD,), jnp.float32)
    zeros2 = jnp.zeros((N_PAD, DH), jnp.float32)

    degp = _sc_degree(dst2d, ones_row, zeros1)
    degT = jnp.transpose(degp.reshape(NC, N_PAD))[:N]
    hp, dinv = _scale(x, W, degT)
    hp2 = hp.reshape(2 * N, DH)
    sp = _sc_scatter(hp2, gidx, dst2d, zeros2)
    return _epilogue(
        sp, hp, dinv, x,
        b.reshape(1, D), gamma.reshape(1, D), beta.reshape(1, D),
    )
